# baseline (speedup 1.0000x reference)
import functools

import jax
import jax.numpy as jnp
from jax import lax
from jax.experimental import pallas as pl
from jax.experimental.pallas import tpu as pltpu
from jax.experimental.pallas import tpu_sc as plsc

NC = 2
NS = 16
L = 16
NW = NC * NS

N = 10000
E = 320000
D = 128
PER_W = E // NW
CHUNK = 80
N_ITERS = PER_W // CHUNK
RS = 8
N_MAIN = 120
OUT_W = 25 * CHUNK
NRM_W = 640
NRM_STEP = 624


def _sc_body(z_hbm, src_hbm, dst_hbm, out_hbm,
             sidx_v, didx_v, rows, out_v, n_v, nsh,
             sem_i0, sem_i1, sem_g1, sem_g2):
    cid = lax.axis_index("c")
    sid = lax.axis_index("s")
    lane = lax.iota(jnp.int32, L)
    base_w = (sid * NC + cid) * PER_W

    ci0 = pltpu.async_copy(src_hbm.at[pl.ds(base_w, PER_W)], sidx_v, sem_i0)
    ci1 = pltpu.async_copy(dst_hbm.at[pl.ds(base_w, PER_W)], didx_v, sem_i1)

    nstart = sid * NRM_STEP

    def sumsq_store(buf, out_off):
        def ngroup(g, carry):
            eids = g * L + lane

            def nd_blk(j, accs):
                a0, a1 = accs
                for u in range(8):
                    dv = (lane + (j * 16 + u)) & (D - 1)
                    v = plsc.load_gather(buf, [eids, dv])
                    a0 = a0 + v * v
                for u in range(8, 16):
                    dv = (lane + (j * 16 + u)) & (D - 1)
                    v = plsc.load_gather(buf, [eids, dv])
                    a1 = a1 + v * v
                return a0, a1

            z2 = jnp.zeros((L,), jnp.float32)
            a0, a1 = lax.fori_loop(0, D // 16, nd_blk, (z2, z2))
            n_v[pl.ds(out_off + g * L, L)] = a0 + a1
            return carry

        lax.fori_loop(0, CHUNK // L, ngroup, 0)

    for c in range(NRM_W // CHUNK):
        pltpu.async_copy(z_hbm.at[pl.ds(nstart + c * CHUNK, CHUNK)],
                         rows[c], sem_g1[c])
    for c in range(NRM_W // CHUNK):
        pltpu.make_async_copy(z_hbm.at[pl.ds(0, CHUNK)], rows[c],
                              sem_g1[c]).wait()
        sumsq_store(rows[c], nstart + c * CHUNK)

    pltpu.sync_copy(n_v.at[pl.ds(nstart, NRM_W)],
                    nsh.at[pl.ds(nstart, NRM_W)])
    plsc.subcore_barrier()
    pltpu.sync_copy(nsh, n_v)

    ci0.wait()
    ci1.wait()

    def issue_g1(b, chunk):
        off = chunk * CHUNK
        pltpu.async_copy(
            z_hbm.at[sidx_v.at[pl.ds(off, CHUNK)]], rows[b], sem_g1[b])

    def wait_g1(b):
        pltpu.make_async_copy(
            z_hbm.at[sidx_v.at[pl.ds(0, CHUNK)]], rows[b], sem_g1[b]).wait()

    def issue_g2(b, chunk):
        off = chunk * CHUNK
        pltpu.async_copy(
            z_hbm.at[didx_v.at[pl.ds(off, CHUNK)]], rows[b], sem_g2[b],
            add=True)

    def wait_g2(b):
        pltpu.make_async_copy(
            z_hbm.at[didx_v.at[pl.ds(0, CHUNK)]], rows[b], sem_g2[b]).wait()

    def compute(b, chunk):
        cbase = (chunk % 25) * CHUNK

        def group_body(g, carry):
            eids = g * L + lane

            def d_blk(j, accs):
                a0, a1 = accs
                for u in range(8):
                    dv = (lane + (j * 16 + u)) & (D - 1)
                    v = plsc.load_gather(rows[b], [eids, dv])
                    a0 = a0 + v * v
                for u in range(8, 16):
                    dv = (lane + (j * 16 + u)) & (D - 1)
                    v = plsc.load_gather(rows[b], [eids, dv])
                    a1 = a1 + v * v
                return a0, a1

            z2 = jnp.zeros((L,), jnp.float32)
            a0, a1 = lax.fori_loop(0, D // 16, d_blk, (z2, z2))
            sq = a0 + a1
            ns = plsc.load_gather(
                n_v, [sidx_v[pl.ds(chunk * CHUNK + g * L, L)]])
            nt = plsc.load_gather(
                n_v, [didx_v[pl.ds(chunk * CHUNK + g * L, L)]])
            val = 0.5 * (sq - ns - nt)
            out_v[pl.ds(cbase + g * L, L)] = 1.0 / (1.0 + jnp.exp(-val))
            return carry

        lax.fori_loop(0, CHUNK // L, group_body, 0)

    for b in range(RS):
        issue_g1(b, b)
    for b in range(4):
        wait_g1(b)
        issue_g2(b, b)

    def step_main(chunk, b):
        wait_g2(b)
        compute(b, chunk)

        @pl.when(chunk + RS < N_ITERS)
        def _g1():
            issue_g1(b, chunk + RS)

        @pl.when(chunk + 4 < N_ITERS)
        def _g2():
            wait_g1((b + 4) % RS)
            issue_g2((b + 4) % RS, chunk + 4)

        @pl.when(chunk % 25 == 24)
        def _flush():
            pltpu.sync_copy(
                out_v,
                out_hbm.at[pl.ds(base_w + (chunk // 25) * OUT_W, OUT_W)])

    def outer(o, carry):
        for b in range(RS):
            step_main(o * RS + b, b)
        return carry

    lax.fori_loop(0, N_MAIN // RS, outer, 0)

    for c in range(N_MAIN, N_ITERS):
        b = c % RS
        wait_g2(b)
        compute(b, c)
        if c + 4 < N_ITERS:
            wait_g1((b + 4) % RS)
            issue_g2((b + 4) % RS, c + 4)
        if c % 25 == 24:
            pltpu.sync_copy(
                out_v,
                out_hbm.at[pl.ds(base_w + (c // 25) * OUT_W, OUT_W)])


@jax.jit
def _run(z, src, dst):
    mesh = plsc.VectorSubcoreMesh(
        core_axis_name="c", subcore_axis_name="s",
        num_cores=NC, num_subcores=NS)
    kfn = pl.kernel(
        _sc_body,
        out_type=jax.ShapeDtypeStruct((E,), jnp.float32),
        mesh=mesh,
        scratch_types=[
            pltpu.VMEM((PER_W,), jnp.int32),
            pltpu.VMEM((PER_W,), jnp.int32),
            [pltpu.VMEM((CHUNK, D), jnp.float32) for _ in range(RS)],
            pltpu.VMEM((OUT_W,), jnp.float32),
            pltpu.VMEM((N,), jnp.float32),
            pltpu.VMEM_SHARED((N,), jnp.float32),
            pltpu.SemaphoreType.DMA,
            pltpu.SemaphoreType.DMA,
            [pltpu.SemaphoreType.DMA for _ in range(RS)],
            [pltpu.SemaphoreType.DMA for _ in range(RS)],
        ],
        compiler_params=pltpu.CompilerParams(needs_layout_passes=False),
    )
    return kfn(z, src, dst)


def kernel(z, edge_index):
    src = edge_index[0].astype(jnp.int32)
    dst = edge_index[1].astype(jnp.int32)
    return _run(z, src, dst)


# speedup vs baseline: 1.0240x; 1.0240x over previous
import functools

import jax
import jax.numpy as jnp
from jax import lax
from jax.experimental import pallas as pl
from jax.experimental.pallas import tpu as pltpu
from jax.experimental.pallas import tpu_sc as plsc

NC = 2
NS = 16
L = 16
NW = NC * NS

E = 320000
D = 128
PER_W = E // NW
CHUNK = 80
N_ITERS = PER_W // CHUNK
N_SLOTS = 5
OUT_W = 5 * N_SLOTS * CHUNK


def _sc_body(z_hbm, src_hbm, dst_hbm, out_hbm, sidx_v, didx_v,
             srows, drows, out_v, sem_i0, sem_i1, sem_s, sem_d):
    wid = lax.axis_index("s") * NC + lax.axis_index("c")
    lane = lax.iota(jnp.int32, L)
    base_w = wid * PER_W

    ci0 = pltpu.async_copy(src_hbm.at[pl.ds(base_w, PER_W)], sidx_v, sem_i0)
    ci1 = pltpu.async_copy(dst_hbm.at[pl.ds(base_w, PER_W)], didx_v, sem_i1)
    ci0.wait()
    ci1.wait()

    def issue(b, chunk):
        off = chunk * CHUNK
        pltpu.async_copy(
            z_hbm.at[sidx_v.at[pl.ds(off, CHUNK)]], srows[b], sem_s[b])
        pltpu.async_copy(
            z_hbm.at[didx_v.at[pl.ds(off, CHUNK)]], drows[b], sem_d[b])

    for b in range(N_SLOTS):
        issue(b, b)

    def compute(b, o, chunk):
        cbase = ((o % 5) * N_SLOTS + (chunk - o * N_SLOTS)) * CHUNK

        def group_body(g, carry):
            eids = g * L + lane

            def d_blk(j, accs):
                a0, a1, a2, a3 = accs
                prods = []
                for u in range(32):
                    dv = (lane + (j * 32 + u)) & (D - 1)
                    s = plsc.load_gather(srows[b], [eids, dv])
                    t = plsc.load_gather(drows[b], [eids, dv])
                    prods.append(s * t)
                for q in range(4):
                    p = prods[q * 8:(q + 1) * 8]
                    tsum = (((p[0] + p[1]) + (p[2] + p[3]))
                            + ((p[4] + p[5]) + (p[6] + p[7])))
                    if q == 0:
                        a0 = a0 + tsum
                    elif q == 1:
                        a1 = a1 + tsum
                    elif q == 2:
                        a2 = a2 + tsum
                    else:
                        a3 = a3 + tsum
                return a0, a1, a2, a3

            z4 = jnp.zeros((L,), jnp.float32)
            a0, a1, a2, a3 = lax.fori_loop(0, D // 32, d_blk,
                                           (z4, z4, z4, z4))
            acc = (a0 + a1) + (a2 + a3)
            out_v[pl.ds(cbase + g * L, L)] = 1.0 / (1.0 + jnp.exp(-acc))
            return carry

        lax.fori_loop(0, CHUNK // L, group_body, 0)

    def outer(o, carry):
        for b in range(N_SLOTS):
            chunk = o * N_SLOTS + b
            pltpu.make_async_copy(
                z_hbm.at[sidx_v.at[pl.ds(0, CHUNK)]], srows[b],
                sem_s[b]).wait()
            pltpu.make_async_copy(
                z_hbm.at[didx_v.at[pl.ds(0, CHUNK)]], drows[b],
                sem_d[b]).wait()
            compute(b, o, chunk)
            nxt = chunk + N_SLOTS

            @pl.when(nxt < N_ITERS)
            def _issue_next():
                issue(b, nxt)

        @pl.when(o % 5 == 4)
        def _flush():
            pltpu.sync_copy(
                out_v, out_hbm.at[pl.ds(base_w + (o // 5) * OUT_W, OUT_W)])

        return carry

    lax.fori_loop(0, N_ITERS // N_SLOTS, outer, 0)


@jax.jit
def _run(z, src, dst):
    mesh = plsc.VectorSubcoreMesh(
        core_axis_name="c", subcore_axis_name="s",
        num_cores=NC, num_subcores=NS)
    kfn = pl.kernel(
        _sc_body,
        out_type=jax.ShapeDtypeStruct((E,), jnp.float32),
        mesh=mesh,
        scratch_types=[
            pltpu.VMEM((PER_W,), jnp.int32),
            pltpu.VMEM((PER_W,), jnp.int32),
            [pltpu.VMEM((CHUNK, D), jnp.float32) for _ in range(N_SLOTS)],
            [pltpu.VMEM((CHUNK, D), jnp.float32) for _ in range(N_SLOTS)],
            pltpu.VMEM((OUT_W,), jnp.float32),
            pltpu.SemaphoreType.DMA,
            pltpu.SemaphoreType.DMA,
            [pltpu.SemaphoreType.DMA for _ in range(N_SLOTS)],
            [pltpu.SemaphoreType.DMA for _ in range(N_SLOTS)],
        ],
        compiler_params=pltpu.CompilerParams(needs_layout_passes=False),
    )
    return kfn(z, src, dst)


def kernel(z, edge_index):
    src = edge_index[0].astype(jnp.int32)
    dst = edge_index[1].astype(jnp.int32)
    return _run(z, src, dst)


# speedup vs baseline: 1.0886x; 1.0630x over previous
import functools

import jax
import jax.numpy as jnp
from jax import lax
from jax.experimental import pallas as pl
from jax.experimental.pallas import tpu as pltpu
from jax.experimental.pallas import tpu_sc as plsc

NC = 2
NS = 16
L = 16
NW = NC * NS

E = 320000
D = 128
PER_W = E // NW
CHUNK = 80
N_ITERS = PER_W // CHUNK
N_SLOTS = 5
OUT_W = 5 * N_SLOTS * CHUNK


def _sc_body(z_hbm, src_hbm, dst_hbm, out_hbm, sidx_v, didx_v,
             srows, drows, out_v, sem_i0, sem_i1, sem_s, sem_d):
    wid = lax.axis_index("s") * NC + lax.axis_index("c")
    lane = lax.iota(jnp.int32, L)
    base_w = wid * PER_W

    ci0 = pltpu.async_copy(src_hbm.at[pl.ds(base_w, PER_W)], sidx_v, sem_i0)
    ci1 = pltpu.async_copy(dst_hbm.at[pl.ds(base_w, PER_W)], didx_v, sem_i1)
    ci0.wait()
    ci1.wait()

    def issue(b, chunk):
        off = chunk * CHUNK
        pltpu.async_copy(
            z_hbm.at[sidx_v.at[pl.ds(off, CHUNK)]], srows[b], sem_s[b])
        pltpu.async_copy(
            z_hbm.at[didx_v.at[pl.ds(off, CHUNK)]], drows[b], sem_d[b])

    for b in range(N_SLOTS):
        issue(b, b)

    def compute(b, o, chunk):
        cbase = ((o % 5) * N_SLOTS + (chunk - o * N_SLOTS)) * CHUNK

        def group_body(g, carry):
            eids = g * L + lane

            def d_blk(j, accs):
                a0, a1, a2, a3 = accs
                prods = []
                for u in range(16):
                    dv = (lane + (j * 16 + u)) & (D - 1)
                    s = plsc.load_gather(srows[b], [eids, dv])
                    t = plsc.load_gather(drows[b], [eids, dv])
                    prods.append(s * t)
                a0 = a0 + ((prods[0] + prods[1]) + (prods[2] + prods[3]))
                a1 = a1 + ((prods[4] + prods[5]) + (prods[6] + prods[7]))
                a2 = a2 + ((prods[8] + prods[9]) + (prods[10] + prods[11]))
                a3 = a3 + ((prods[12] + prods[13]) + (prods[14] + prods[15]))
                return a0, a1, a2, a3

            z4 = jnp.zeros((L,), jnp.float32)
            a0, a1, a2, a3 = lax.fori_loop(0, D // 16, d_blk,
                                           (z4, z4, z4, z4))
            acc = (a0 + a1) + (a2 + a3)
            out_v[pl.ds(cbase + g * L, L)] = 1.0 / (1.0 + jnp.exp(-acc))
            return carry

        lax.fori_loop(0, CHUNK // L, group_body, 0)

    def outer(o, carry):
        for b in range(N_SLOTS):
            chunk = o * N_SLOTS + b
            pltpu.make_async_copy(
                z_hbm.at[sidx_v.at[pl.ds(0, CHUNK)]], srows[b],
                sem_s[b]).wait()
            pltpu.make_async_copy(
                z_hbm.at[didx_v.at[pl.ds(0, CHUNK)]], drows[b],
                sem_d[b]).wait()
            compute(b, o, chunk)
            nxt = chunk + N_SLOTS

            @pl.when(nxt < N_ITERS)
            def _issue_next():
                issue(b, nxt)

        @pl.when(o % 5 == 4)
        def _flush():
            pltpu.sync_copy(
                out_v, out_hbm.at[pl.ds(base_w + (o // 5) * OUT_W, OUT_W)])

        return carry

    lax.fori_loop(0, N_ITERS // N_SLOTS, outer, 0)


@jax.jit
def _run(z, src, dst):
    mesh = plsc.VectorSubcoreMesh(
        core_axis_name="c", subcore_axis_name="s",
        num_cores=NC, num_subcores=NS)
    kfn = pl.kernel(
        _sc_body,
        out_type=jax.ShapeDtypeStruct((E,), jnp.float32),
        mesh=mesh,
        scratch_types=[
            pltpu.VMEM((PER_W,), jnp.int32),
            pltpu.VMEM((PER_W,), jnp.int32),
            [pltpu.VMEM((CHUNK, D), jnp.float32) for _ in range(N_SLOTS)],
            [pltpu.VMEM((CHUNK, D), jnp.float32) for _ in range(N_SLOTS)],
            pltpu.VMEM((OUT_W,), jnp.float32),
            pltpu.SemaphoreType.DMA,
            pltpu.SemaphoreType.DMA,
            [pltpu.SemaphoreType.DMA for _ in range(N_SLOTS)],
            [pltpu.SemaphoreType.DMA for _ in range(N_SLOTS)],
        ],
        compiler_params=pltpu.CompilerParams(needs_layout_passes=False),
    )
    return kfn(z, src, dst)


def kernel(z, edge_index):
    src = edge_index[0].astype(jnp.int32)
    dst = edge_index[1].astype(jnp.int32)
    return _run(z, src, dst)
